# Initial kernel scaffold; baseline (speedup 1.0000x reference)
import functools

import jax
import jax.numpy as jnp
from jax import lax
from jax.experimental import pallas as pl
from jax.experimental.pallas import tpu as pltpu
from jax.experimental.pallas import tpu_sc as plsc

N = 10000
E = 320000
D = 128

NC = 2
NS = 16
L = 16
NW = NC * NS
EPT = E // NW
CHUNK = 80
NCHUNKS = EPT // CHUNK
RPT = N // NS
AW = D + L

_mesh = plsc.VectorSubcoreMesh(core_axis_name="c", subcore_axis_name="s")


@functools.partial(
    pl.kernel,
    out_type=jax.ShapeDtypeStruct((NC, N, AW), jnp.float32),
    mesh=_mesh,
    scratch_types=[
        pltpu.VMEM((CHUNK,), jnp.int32),
        pltpu.VMEM((CHUNK,), jnp.int32),
        pltpu.SMEM((CHUNK,), jnp.float32),
        pltpu.VMEM((CHUNK, D), jnp.float32),
        pltpu.VMEM((CHUNK, AW), jnp.float32),
        pltpu.VMEM_SHARED((N, AW), jnp.float32),
        pltpu.SemaphoreType.DMA,
    ],
)
def _sc_agg(y_hbm, src_hbm, dst_hbm, w_hbm, out_hbm,
            srci, dsti, w_s, grows, srows, acc, sem):
    c = lax.axis_index("c")
    s = lax.axis_index("s")
    wid = s * NC + c

    zv = jnp.zeros((L,), jnp.float32)

    @pl.loop(0, CHUNK)
    def _(r):
        for q in range(AW // L):
            srows[r, pl.ds(q * L, L)] = zv

    row0 = s * RPT
    for k in range(RPT // CHUNK):
        pltpu.sync_copy(srows, acc.at[pl.ds(row0 + k * CHUNK, CHUNK)])
    rem = RPT % CHUNK
    pltpu.sync_copy(srows.at[pl.ds(0, rem)],
                    acc.at[pl.ds(row0 + (RPT // CHUNK) * CHUNK, rem)])

    tailv = jnp.where(lax.iota(jnp.int32, L) == 0, 1.0, 0.0).astype(jnp.float32)

    @pl.loop(0, CHUNK)
    def _(r):
        srows[r, pl.ds(D, L)] = tailv

    plsc.subcore_barrier()

    ebase = wid * EPT

    @pl.loop(0, NCHUNKS)
    def _(cidx):
        base = ebase + cidx * CHUNK
        pltpu.sync_copy(src_hbm.at[pl.ds(base, CHUNK)], srci)
        pltpu.sync_copy(w_hbm.at[pl.ds(base, CHUNK)], w_s)
        pltpu.sync_copy(dst_hbm.at[pl.ds(base, CHUNK)], dsti)
        pltpu.async_copy(y_hbm.at[srci], grows, sem).wait()

        @pl.loop(0, CHUNK)
        def _(j):
            wv = jnp.full((L,), w_s[j], jnp.float32)
            for q in range(D // L):
                sl = pl.ds(q * L, L)
                srows[j, sl] = grows[j, sl] * wv

        pltpu.sync_copy(srows, acc.at[dsti], add=True)

    plsc.subcore_barrier()
    pltpu.sync_copy(acc.at[pl.ds(row0, RPT)], out_hbm.at[c, pl.ds(row0, RPT)])


_BLK = 2000
_GRID = N // _BLK


def _tc_pre_body(x_ref, wl_ref, wr_ref, b_ref, y_ref, r_ref):
    xb = x_ref[...]
    y_ref[...] = jnp.dot(xb, wl_ref[...], preferred_element_type=jnp.float32)
    r_ref[...] = (jnp.dot(xb, wr_ref[...], preferred_element_type=jnp.float32)
                  + b_ref[...])


_tc_pre = pl.pallas_call(
    _tc_pre_body,
    grid=(_GRID,),
    in_specs=[
        pl.BlockSpec((_BLK, D), lambda i: (i, 0)),
        pl.BlockSpec((D, D), lambda i: (0, 0)),
        pl.BlockSpec((D, D), lambda i: (0, 0)),
        pl.BlockSpec((1, D), lambda i: (0, 0)),
    ],
    out_specs=[
        pl.BlockSpec((_BLK, D), lambda i: (i, 0)),
        pl.BlockSpec((_BLK, D), lambda i: (i, 0)),
    ],
    out_shape=[
        jax.ShapeDtypeStruct((N, D), jnp.float32),
        jax.ShapeDtypeStruct((N, D), jnp.float32),
    ],
)


def _tc_mid_body(p_ref, r1_ref, wl_ref, wr_ref, b_ref, y_ref, r_ref):
    z = p_ref[0] + p_ref[1]
    cnt = jnp.clip(z[:, D:D + 1], 1.0, None)
    h = jnp.maximum(z[:, :D] / cnt + r1_ref[...], 0.0)
    y_ref[...] = jnp.dot(h, wl_ref[...], preferred_element_type=jnp.float32)
    r_ref[...] = (jnp.dot(h, wr_ref[...], preferred_element_type=jnp.float32)
                  + b_ref[...])


_tc_mid = pl.pallas_call(
    _tc_mid_body,
    grid=(_GRID,),
    in_specs=[
        pl.BlockSpec((NC, _BLK, AW), lambda i: (0, i, 0)),
        pl.BlockSpec((_BLK, D), lambda i: (i, 0)),
        pl.BlockSpec((D, D), lambda i: (0, 0)),
        pl.BlockSpec((D, D), lambda i: (0, 0)),
        pl.BlockSpec((1, D), lambda i: (0, 0)),
    ],
    out_specs=[
        pl.BlockSpec((_BLK, D), lambda i: (i, 0)),
        pl.BlockSpec((_BLK, D), lambda i: (i, 0)),
    ],
    out_shape=[
        jax.ShapeDtypeStruct((N, D), jnp.float32),
        jax.ShapeDtypeStruct((N, D), jnp.float32),
    ],
)


def _tc_post_body(p_ref, r2_ref, o_ref):
    z = p_ref[0] + p_ref[1]
    cnt = jnp.clip(z[:, D:D + 1], 1.0, None)
    o_ref[...] = z[:, :D] / cnt + r2_ref[...]


_tc_post = pl.pallas_call(
    _tc_post_body,
    grid=(_GRID,),
    in_specs=[
        pl.BlockSpec((NC, _BLK, AW), lambda i: (0, i, 0)),
        pl.BlockSpec((_BLK, D), lambda i: (i, 0)),
    ],
    out_specs=pl.BlockSpec((_BLK, D), lambda i: (i, 0)),
    out_shape=jax.ShapeDtypeStruct((N, D), jnp.float32),
)


def kernel(x, edge_index, edge_weight, W1l, W1r, b1, W2l, W2r, b2):
    src = edge_index[0]
    dst = edge_index[1]
    y1, r1 = _tc_pre(x, W1l, W1r, b1.reshape(1, D))
    p1 = _sc_agg(y1, src, dst, edge_weight)
    y2, r2 = _tc_mid(p1, r1, W2l, W2r, b2.reshape(1, D))
    p2 = _sc_agg(y2, src, dst, edge_weight)
    return _tc_post(p2, r2)


# speedup vs baseline: 2.3984x; 2.3984x over previous
import functools

import jax
import jax.numpy as jnp
from jax import lax
from jax.experimental import pallas as pl
from jax.experimental.pallas import tpu as pltpu
from jax.experimental.pallas import tpu_sc as plsc

N = 10000
E = 320000
D = 128

NC = 2
NS = 16
L = 16
NW = NC * NS
EPT = E // NW
CHUNK = 80
NCHUNKS = EPT // CHUNK
NPAD = 10240
RPT = NPAD // NS
AW = D + L

_mesh = plsc.VectorSubcoreMesh(core_axis_name="c", subcore_axis_name="s")


def _splat_lane(vec, lane):
    idx = jnp.full((L, 1), lane, jnp.int32)
    dnums = lax.GatherDimensionNumbers(
        offset_dims=(), collapsed_slice_dims=(0,), start_index_map=(0,))
    return lax.gather(vec, idx, dnums, slice_sizes=(1,),
                      mode=lax.GatherScatterMode.PROMISE_IN_BOUNDS)


@functools.partial(
    pl.kernel,
    out_type=jax.ShapeDtypeStruct((NC, NPAD, AW), jnp.float32),
    mesh=_mesh,
    scratch_types=[
        pltpu.VMEM((CHUNK,), jnp.int32),
        pltpu.VMEM((CHUNK,), jnp.int32),
        pltpu.VMEM((CHUNK,), jnp.float32),
        pltpu.VMEM((CHUNK, D), jnp.float32),
        pltpu.VMEM((CHUNK, AW), jnp.float32),
        pltpu.VMEM_SHARED((NPAD, AW), jnp.float32),
        pltpu.SemaphoreType.DMA,
    ],
    compiler_params=pltpu.CompilerParams(use_tc_tiling_on_sc=False),
)
def _sc_agg(y_hbm, src_hbm, dst_hbm, w_hbm, out_hbm,
            srci, dsti, w_v, grows, srows, acc, sem):
    c = lax.axis_index("c")
    s = lax.axis_index("s")
    wid = s * NC + c

    zv = jnp.zeros((L,), jnp.float32)

    @pl.loop(0, CHUNK)
    def _(r):
        for q in range(AW // L):
            srows[r, pl.ds(q * L, L)] = zv

    row0 = s * RPT
    for k in range(RPT // CHUNK):
        pltpu.sync_copy(srows, acc.at[pl.ds(row0 + k * CHUNK, CHUNK)])

    tailv = jnp.where(lax.iota(jnp.int32, L) == 0, 1.0, 0.0).astype(jnp.float32)

    @pl.loop(0, CHUNK)
    def _(r):
        srows[r, pl.ds(D, L)] = tailv

    plsc.subcore_barrier()

    ebase = wid * EPT

    @pl.loop(0, NCHUNKS)
    def _(cidx):
        base = ebase + cidx * CHUNK
        pltpu.sync_copy(src_hbm.at[pl.ds(base, CHUNK)], srci)
        pltpu.sync_copy(w_hbm.at[pl.ds(base, CHUNK)], w_v)
        pltpu.sync_copy(dst_hbm.at[pl.ds(base, CHUNK)], dsti)
        pltpu.async_copy(y_hbm.at[srci], grows, sem).wait()

        @pl.loop(0, CHUNK // L)
        def _(g):
            wvec = w_v[pl.ds(g * L, L)]
            for j2 in range(L):
                ws = _splat_lane(wvec, j2)
                j = g * L + j2
                for q in range(D // L):
                    sl = pl.ds(q * L, L)
                    srows[j, sl] = grows[j, sl] * ws

        pltpu.sync_copy(srows, acc.at[dsti], add=True)

    plsc.subcore_barrier()
    pltpu.sync_copy(acc.at[pl.ds(row0, RPT)], out_hbm.at[c, pl.ds(row0, RPT)])


_BLK = 2000
_GRID = N // _BLK


def _tc_pre_body(x_ref, wl_ref, wr_ref, b_ref, y_ref, r_ref):
    xb = x_ref[...]
    y_ref[...] = jnp.dot(xb, wl_ref[...], preferred_element_type=jnp.float32)
    r_ref[...] = (jnp.dot(xb, wr_ref[...], preferred_element_type=jnp.float32)
                  + b_ref[...])


_tc_pre = pl.pallas_call(
    _tc_pre_body,
    grid=(_GRID,),
    in_specs=[
        pl.BlockSpec((_BLK, D), lambda i: (i, 0)),
        pl.BlockSpec((D, D), lambda i: (0, 0)),
        pl.BlockSpec((D, D), lambda i: (0, 0)),
        pl.BlockSpec((1, D), lambda i: (0, 0)),
    ],
    out_specs=[
        pl.BlockSpec((_BLK, D), lambda i: (i, 0)),
        pl.BlockSpec((_BLK, D), lambda i: (i, 0)),
    ],
    out_shape=[
        jax.ShapeDtypeStruct((N, D), jnp.float32),
        jax.ShapeDtypeStruct((N, D), jnp.float32),
    ],
)


def _tc_mid_body(p_ref, r1_ref, wl_ref, wr_ref, b_ref, y_ref, r_ref):
    z = p_ref[0] + p_ref[1]
    cnt = jnp.clip(z[:, D:D + 1], 1.0, None)
    h = jnp.maximum(z[:, :D] / cnt + r1_ref[...], 0.0)
    y_ref[...] = jnp.dot(h, wl_ref[...], preferred_element_type=jnp.float32)
    r_ref[...] = (jnp.dot(h, wr_ref[...], preferred_element_type=jnp.float32)
                  + b_ref[...])


_tc_mid = pl.pallas_call(
    _tc_mid_body,
    grid=(_GRID,),
    in_specs=[
        pl.BlockSpec((NC, _BLK, AW), lambda i: (0, i, 0)),
        pl.BlockSpec((_BLK, D), lambda i: (i, 0)),
        pl.BlockSpec((D, D), lambda i: (0, 0)),
        pl.BlockSpec((D, D), lambda i: (0, 0)),
        pl.BlockSpec((1, D), lambda i: (0, 0)),
    ],
    out_specs=[
        pl.BlockSpec((_BLK, D), lambda i: (i, 0)),
        pl.BlockSpec((_BLK, D), lambda i: (i, 0)),
    ],
    out_shape=[
        jax.ShapeDtypeStruct((N, D), jnp.float32),
        jax.ShapeDtypeStruct((N, D), jnp.float32),
    ],
)


def _tc_post_body(p_ref, r2_ref, o_ref):
    z = p_ref[0] + p_ref[1]
    cnt = jnp.clip(z[:, D:D + 1], 1.0, None)
    o_ref[...] = z[:, :D] / cnt + r2_ref[...]


_tc_post = pl.pallas_call(
    _tc_post_body,
    grid=(_GRID,),
    in_specs=[
        pl.BlockSpec((NC, _BLK, AW), lambda i: (0, i, 0)),
        pl.BlockSpec((_BLK, D), lambda i: (i, 0)),
    ],
    out_specs=pl.BlockSpec((_BLK, D), lambda i: (i, 0)),
    out_shape=jax.ShapeDtypeStruct((N, D), jnp.float32),
)


def kernel(x, edge_index, edge_weight, W1l, W1r, b1, W2l, W2r, b2):
    src = edge_index[0]
    dst = edge_index[1]
    y1, r1 = _tc_pre(x, W1l, W1r, b1.reshape(1, D))
    p1 = _sc_agg(y1, src, dst, edge_weight)
    y2, r2 = _tc_mid(p1, r1, W2l, W2r, b2.reshape(1, D))
    p2 = _sc_agg(y2, src, dst, edge_weight)
    return _tc_post(p2, r2)


# speedup vs baseline: 3.3987x; 1.4170x over previous
import functools

import jax
import jax.numpy as jnp
from jax import lax
from jax.experimental import pallas as pl
from jax.experimental.pallas import tpu as pltpu
from jax.experimental.pallas import tpu_sc as plsc

N = 10000
E = 320000
D = 128

NC = 2
NS = 16
L = 16
NW = NC * NS
EPT = E // NW
CHUNK = 80
NCHUNKS = EPT // CHUNK
NPAD = 10240
RPT = NPAD // NS
AW = D + L

_mesh = plsc.VectorSubcoreMesh(core_axis_name="c", subcore_axis_name="s")


def _splat_lane(vec, lane):
    idx = jnp.full((L, 1), lane, jnp.int32)
    dnums = lax.GatherDimensionNumbers(
        offset_dims=(), collapsed_slice_dims=(0,), start_index_map=(0,))
    return lax.gather(vec, idx, dnums, slice_sizes=(1,),
                      mode=lax.GatherScatterMode.PROMISE_IN_BOUNDS)


@functools.partial(
    pl.kernel,
    out_type=jax.ShapeDtypeStruct((NC, NPAD, AW), jnp.float32),
    mesh=_mesh,
    scratch_types=[
        pltpu.VMEM((CHUNK,), jnp.int32),
        pltpu.VMEM((CHUNK,), jnp.int32),
        pltpu.VMEM((CHUNK,), jnp.float32),
        pltpu.VMEM((CHUNK,), jnp.int32),
        pltpu.VMEM((CHUNK,), jnp.int32),
        pltpu.VMEM((CHUNK,), jnp.float32),
        pltpu.VMEM((CHUNK, D), jnp.float32),
        pltpu.VMEM((CHUNK, D), jnp.float32),
        pltpu.VMEM((CHUNK, AW), jnp.float32),
        pltpu.VMEM_SHARED((NPAD, AW), jnp.float32),
        pltpu.SemaphoreType.DMA,
        pltpu.SemaphoreType.DMA,
        pltpu.SemaphoreType.DMA,
        pltpu.SemaphoreType.DMA,
    ],
    compiler_params=pltpu.CompilerParams(use_tc_tiling_on_sc=False),
)
def _sc_agg(y_hbm, src_hbm, dst_hbm, w_hbm, out_hbm,
            srci0, dsti0, wv0, srci1, dsti1, wv1, grows0, grows1, srows,
            acc, si0, si1, sg0, sg1):
    c = lax.axis_index("c")
    s = lax.axis_index("s")
    wid = s * NC + c
    ebase = wid * EPT

    zv = jnp.zeros((L,), jnp.float32)

    @pl.loop(0, CHUNK)
    def _(r):
        for q in range(AW // L):
            srows[r, pl.ds(q * L, L)] = zv

    row0 = s * RPT
    for k in range(RPT // CHUNK):
        pltpu.sync_copy(srows, acc.at[pl.ds(row0 + k * CHUNK, CHUNK)])

    tailv = jnp.where(lax.iota(jnp.int32, L) == 0, 1.0, 0.0).astype(jnp.float32)

    @pl.loop(0, CHUNK)
    def _(r):
        srows[r, pl.ds(D, L)] = tailv

    plsc.subcore_barrier()

    def _idx_copies(cidx, si, di, wv, sem):
        base = ebase + cidx * CHUNK
        return (
            pltpu.make_async_copy(src_hbm.at[pl.ds(base, CHUNK)], si, sem),
            pltpu.make_async_copy(dst_hbm.at[pl.ds(base, CHUNK)], di, sem),
            pltpu.make_async_copy(w_hbm.at[pl.ds(base, CHUNK)], wv, sem),
        )

    def _idx_issue(cidx, si, di, wv, sem):
        for cp in _idx_copies(cidx, si, di, wv, sem):
            cp.start()

    def _idx_wait(cidx, si, di, wv, sem):
        for cp in _idx_copies(cidx, si, di, wv, sem):
            cp.wait()

    def _gather(si, gbuf, sem):
        return pltpu.make_async_copy(y_hbm.at[si], gbuf, sem)

    def _process(gbuf, wv, di):
        @pl.loop(0, CHUNK // L)
        def _(g):
            wvec = wv[pl.ds(g * L, L)]
            for j2 in range(L):
                ws = _splat_lane(wvec, j2)
                j = g * L + j2
                for q in range(D // L):
                    sl = pl.ds(q * L, L)
                    srows[j, sl] = gbuf[j, sl] * ws

        pltpu.sync_copy(srows, acc.at[di], add=True)

    _idx_issue(0, srci0, dsti0, wv0, si0)
    _idx_issue(1, srci1, dsti1, wv1, si1)
    _idx_wait(0, srci0, dsti0, wv0, si0)
    _gather(srci0, grows0, sg0).start()

    @pl.loop(0, NCHUNKS - 4, step=2)
    def _(i):
        _idx_wait(i + 1, srci1, dsti1, wv1, si1)
        _gather(srci1, grows1, sg1).start()
        _gather(srci0, grows0, sg0).wait()
        _process(grows0, wv0, dsti0)
        _idx_issue(i + 2, srci0, dsti0, wv0, si0)
        _idx_wait(i + 2, srci0, dsti0, wv0, si0)
        _gather(srci0, grows0, sg0).start()
        _gather(srci1, grows1, sg1).wait()
        _process(grows1, wv1, dsti1)
        _idx_issue(i + 3, srci1, dsti1, wv1, si1)

    _idx_wait(NCHUNKS - 2, srci1, dsti1, wv1, si1)
    _gather(srci1, grows1, sg1).start()
    _gather(srci0, grows0, sg0).wait()
    _process(grows0, wv0, dsti0)
    _idx_issue(NCHUNKS - 1, srci0, dsti0, wv0, si0)
    _idx_wait(NCHUNKS - 1, srci0, dsti0, wv0, si0)
    _gather(srci0, grows0, sg0).start()
    _gather(srci1, grows1, sg1).wait()
    _process(grows1, wv1, dsti1)
    _gather(srci0, grows0, sg0).wait()
    _process(grows0, wv0, dsti0)

    plsc.subcore_barrier()
    pltpu.sync_copy(acc.at[pl.ds(row0, RPT)], out_hbm.at[c, pl.ds(row0, RPT)])


_BLK = 2000
_GRID = N // _BLK


def _tc_pre_body(x_ref, wl_ref, wr_ref, b_ref, y_ref, r_ref):
    xb = x_ref[...]
    y_ref[...] = jnp.dot(xb, wl_ref[...], preferred_element_type=jnp.float32)
    r_ref[...] = (jnp.dot(xb, wr_ref[...], preferred_element_type=jnp.float32)
                  + b_ref[...])


_tc_pre = pl.pallas_call(
    _tc_pre_body,
    grid=(_GRID,),
    in_specs=[
        pl.BlockSpec((_BLK, D), lambda i: (i, 0)),
        pl.BlockSpec((D, D), lambda i: (0, 0)),
        pl.BlockSpec((D, D), lambda i: (0, 0)),
        pl.BlockSpec((1, D), lambda i: (0, 0)),
    ],
    out_specs=[
        pl.BlockSpec((_BLK, D), lambda i: (i, 0)),
        pl.BlockSpec((_BLK, D), lambda i: (i, 0)),
    ],
    out_shape=[
        jax.ShapeDtypeStruct((N, D), jnp.float32),
        jax.ShapeDtypeStruct((N, D), jnp.float32),
    ],
)


def _tc_mid_body(p_ref, r1_ref, wl_ref, wr_ref, b_ref, y_ref, r_ref):
    z = p_ref[0] + p_ref[1]
    cnt = jnp.clip(z[:, D:D + 1], 1.0, None)
    h = jnp.maximum(z[:, :D] / cnt + r1_ref[...], 0.0)
    y_ref[...] = jnp.dot(h, wl_ref[...], preferred_element_type=jnp.float32)
    r_ref[...] = (jnp.dot(h, wr_ref[...], preferred_element_type=jnp.float32)
                  + b_ref[...])


_tc_mid = pl.pallas_call(
    _tc_mid_body,
    grid=(_GRID,),
    in_specs=[
        pl.BlockSpec((NC, _BLK, AW), lambda i: (0, i, 0)),
        pl.BlockSpec((_BLK, D), lambda i: (i, 0)),
        pl.BlockSpec((D, D), lambda i: (0, 0)),
        pl.BlockSpec((D, D), lambda i: (0, 0)),
        pl.BlockSpec((1, D), lambda i: (0, 0)),
    ],
    out_specs=[
        pl.BlockSpec((_BLK, D), lambda i: (i, 0)),
        pl.BlockSpec((_BLK, D), lambda i: (i, 0)),
    ],
    out_shape=[
        jax.ShapeDtypeStruct((N, D), jnp.float32),
        jax.ShapeDtypeStruct((N, D), jnp.float32),
    ],
)


def _tc_post_body(p_ref, r2_ref, o_ref):
    z = p_ref[0] + p_ref[1]
    cnt = jnp.clip(z[:, D:D + 1], 1.0, None)
    o_ref[...] = z[:, :D] / cnt + r2_ref[...]


_tc_post = pl.pallas_call(
    _tc_post_body,
    grid=(_GRID,),
    in_specs=[
        pl.BlockSpec((NC, _BLK, AW), lambda i: (0, i, 0)),
        pl.BlockSpec((_BLK, D), lambda i: (i, 0)),
    ],
    out_specs=pl.BlockSpec((_BLK, D), lambda i: (i, 0)),
    out_shape=jax.ShapeDtypeStruct((N, D), jnp.float32),
)


def kernel(x, edge_index, edge_weight, W1l, W1r, b1, W2l, W2r, b2):
    src = edge_index[0]
    dst = edge_index[1]
    y1, r1 = _tc_pre(x, W1l, W1r, b1.reshape(1, D))
    p1 = _sc_agg(y1, src, dst, edge_weight)
    y2, r2 = _tc_mid(p1, r1, W2l, W2r, b2.reshape(1, D))
    p2 = _sc_agg(y2, src, dst, edge_weight)
    return _tc_post(p2, r2)


# speedup vs baseline: 10.0822x; 2.9665x over previous
import functools

import jax
import jax.numpy as jnp
from jax import lax
from jax.experimental import pallas as pl
from jax.experimental.pallas import tpu as pltpu
from jax.experimental.pallas import tpu_sc as plsc

N = 10000
E = 320000
D = 128

NC = 2
NS = 16
L = 16
NW = NC * NS
EPT = E // NW
CHUNK = 80
NCHUNKS = EPT // CHUNK
NPAD = 10240
RPT = NPAD // NS

_mesh = plsc.VectorSubcoreMesh(core_axis_name="c", subcore_axis_name="s")


def _splat_lane(vec, lane):
    idx = jnp.full((L, 1), lane, jnp.int32)
    dnums = lax.GatherDimensionNumbers(
        offset_dims=(), collapsed_slice_dims=(0,), start_index_map=(0,))
    return lax.gather(vec, idx, dnums, slice_sizes=(1,),
                      mode=lax.GatherScatterMode.PROMISE_IN_BOUNDS)


def _idx_scratch():
    return [
        pltpu.VMEM((CHUNK,), jnp.int32),
        pltpu.VMEM((CHUNK,), jnp.int32),
        pltpu.VMEM((CHUNK,), jnp.float32),
        pltpu.VMEM((CHUNK, D), jnp.float32),
        pltpu.SemaphoreType.DMA,
        pltpu.SemaphoreType.DMA,
        pltpu.SemaphoreType.DMA,
    ]


@functools.partial(
    pl.kernel,
    out_type=(
        jax.ShapeDtypeStruct((NC, NPAD, D), jnp.float32),
        jax.ShapeDtypeStruct((NC, NPAD, L), jnp.float32),
    ),
    mesh=_mesh,
    scratch_types=[
        *_idx_scratch(), *_idx_scratch(), *_idx_scratch(),
        pltpu.VMEM((CHUNK, L), jnp.float32),
        pltpu.VMEM_SHARED((NPAD, D), jnp.float32),
        pltpu.VMEM_SHARED((NPAD, L), jnp.float32),
    ],
    compiler_params=pltpu.CompilerParams(use_tc_tiling_on_sc=False),
)
def _sc_agg(y_hbm, src_hbm, dst_hbm, w_hbm, outx_hbm, outc_hbm,
            sa, da, wa, ga, ia, gsa, ssa,
            sb, db, wb, gb, ib, gsb, ssb,
            sc_, dc, wc, gc, ic, gsc, ssc,
            ones, accx, accc):
    c = lax.axis_index("c")
    s = lax.axis_index("s")
    wid = s * NC + c
    ebase = wid * EPT
    row0 = s * RPT

    A = (sa, da, wa, ga, ia, gsa, ssa)
    B = (sb, db, wb, gb, ib, gsb, ssb)
    C = (sc_, dc, wc, gc, ic, gsc, ssc)

    zv = jnp.zeros((L,), jnp.float32)

    @pl.loop(0, CHUNK)
    def _(r):
        for q in range(D // L):
            ga[r, pl.ds(q * L, L)] = zv
        ones[r, pl.ds(0, L)] = zv

    for k in range(RPT // CHUNK):
        pltpu.sync_copy(ga, accx.at[pl.ds(row0 + k * CHUNK, CHUNK)])
        pltpu.sync_copy(ones, accc.at[pl.ds(row0 + k * CHUNK, CHUNK)])

    tailv = jnp.where(lax.iota(jnp.int32, L) == 0, 1.0, 0.0).astype(jnp.float32)

    @pl.loop(0, CHUNK)
    def _(r):
        ones[r, pl.ds(0, L)] = tailv

    plsc.subcore_barrier()

    def _prep(cidx, buf, first=False):
        si, di, wv, gbuf, isem, gsem, ssem = buf
        if not first:
            pltpu.make_async_copy(gbuf, accx.at[di], ssem).wait()
            pltpu.make_async_copy(ones, accc.at[di], ssem).wait()
        base = ebase + cidx * CHUNK
        pltpu.make_async_copy(src_hbm.at[pl.ds(base, CHUNK)], si, isem).start()
        pltpu.make_async_copy(dst_hbm.at[pl.ds(base, CHUNK)], di, isem).start()
        pltpu.make_async_copy(w_hbm.at[pl.ds(base, CHUNK)], wv, isem).start()
        pltpu.make_async_copy(src_hbm.at[pl.ds(base, CHUNK)], si, isem).wait()
        pltpu.make_async_copy(dst_hbm.at[pl.ds(base, CHUNK)], di, isem).wait()
        pltpu.make_async_copy(w_hbm.at[pl.ds(base, CHUNK)], wv, isem).wait()
        pltpu.make_async_copy(y_hbm.at[si], gbuf, gsem).start()

    def _process(buf):
        si, di, wv, gbuf, isem, gsem, ssem = buf
        pltpu.make_async_copy(y_hbm.at[si], gbuf, gsem).wait()

        @pl.loop(0, CHUNK // L)
        def _(g):
            wvec = wv[pl.ds(g * L, L)]
            for j2 in range(L):
                ws = _splat_lane(wvec, j2)
                j = g * L + j2
                for q in range(D // L):
                    sl = pl.ds(q * L, L)
                    gbuf[j, sl] = gbuf[j, sl] * ws

        pltpu.async_copy(gbuf, accx.at[di], ssem, add=True)
        pltpu.async_copy(ones, accc.at[di], ssem, add=True)

    _prep(0, A, first=True)
    _prep(1, B, first=True)
    _process(A)
    _prep(2, C, first=True)
    _process(B)
    _prep(3, A)

    @pl.loop(2, NCHUNKS - 3, step=3)
    def _(x):
        _process(C)
        _prep(x + 2, B)
        _process(A)
        _prep(x + 3, C)
        _process(B)
        _prep(x + 4, A)

    _process(C)
    _prep(NCHUNKS - 1, B)
    _process(A)
    _process(B)

    for buf in (C, A, B):
        si, di, wv, gbuf, isem, gsem, ssem = buf
        pltpu.make_async_copy(gbuf, accx.at[di], ssem).wait()
        pltpu.make_async_copy(ones, accc.at[di], ssem).wait()

    plsc.subcore_barrier()
    pltpu.sync_copy(accx.at[pl.ds(row0, RPT)], outx_hbm.at[c, pl.ds(row0, RPT)])
    pltpu.sync_copy(accc.at[pl.ds(row0, RPT)], outc_hbm.at[c, pl.ds(row0, RPT)])


_BLK = 2000
_GRID = N // _BLK


def _tc_pre_body(x_ref, wl_ref, wr_ref, b_ref, y_ref, r_ref):
    xb = x_ref[...]
    y_ref[...] = jnp.dot(xb, wl_ref[...], preferred_element_type=jnp.float32)
    r_ref[...] = (jnp.dot(xb, wr_ref[...], preferred_element_type=jnp.float32)
                  + b_ref[...])


_tc_pre = pl.pallas_call(
    _tc_pre_body,
    grid=(_GRID,),
    in_specs=[
        pl.BlockSpec((_BLK, D), lambda i: (i, 0)),
        pl.BlockSpec((D, D), lambda i: (0, 0)),
        pl.BlockSpec((D, D), lambda i: (0, 0)),
        pl.BlockSpec((1, D), lambda i: (0, 0)),
    ],
    out_specs=[
        pl.BlockSpec((_BLK, D), lambda i: (i, 0)),
        pl.BlockSpec((_BLK, D), lambda i: (i, 0)),
    ],
    out_shape=[
        jax.ShapeDtypeStruct((N, D), jnp.float32),
        jax.ShapeDtypeStruct((N, D), jnp.float32),
    ],
)


def _mean_agg(px_ref, pc_ref):
    z = px_ref[0] + px_ref[1]
    cnt = pc_ref[0, :, 0:1] + pc_ref[1, :, 0:1]
    return z / jnp.clip(cnt, 1.0, None)


def _tc_mid_body(px_ref, pc_ref, r1_ref, wl_ref, wr_ref, b_ref, y_ref, r_ref):
    h = jnp.maximum(_mean_agg(px_ref, pc_ref) + r1_ref[...], 0.0)
    y_ref[...] = jnp.dot(h, wl_ref[...], preferred_element_type=jnp.float32)
    r_ref[...] = (jnp.dot(h, wr_ref[...], preferred_element_type=jnp.float32)
                  + b_ref[...])


_tc_mid = pl.pallas_call(
    _tc_mid_body,
    grid=(_GRID,),
    in_specs=[
        pl.BlockSpec((NC, _BLK, D), lambda i: (0, i, 0)),
        pl.BlockSpec((NC, _BLK, L), lambda i: (0, i, 0)),
        pl.BlockSpec((_BLK, D), lambda i: (i, 0)),
        pl.BlockSpec((D, D), lambda i: (0, 0)),
        pl.BlockSpec((D, D), lambda i: (0, 0)),
        pl.BlockSpec((1, D), lambda i: (0, 0)),
    ],
    out_specs=[
        pl.BlockSpec((_BLK, D), lambda i: (i, 0)),
        pl.BlockSpec((_BLK, D), lambda i: (i, 0)),
    ],
    out_shape=[
        jax.ShapeDtypeStruct((N, D), jnp.float32),
        jax.ShapeDtypeStruct((N, D), jnp.float32),
    ],
)


def _tc_post_body(px_ref, pc_ref, r2_ref, o_ref):
    o_ref[...] = _mean_agg(px_ref, pc_ref) + r2_ref[...]


_tc_post = pl.pallas_call(
    _tc_post_body,
    grid=(_GRID,),
    in_specs=[
        pl.BlockSpec((NC, _BLK, D), lambda i: (0, i, 0)),
        pl.BlockSpec((NC, _BLK, L), lambda i: (0, i, 0)),
        pl.BlockSpec((_BLK, D), lambda i: (i, 0)),
    ],
    out_specs=pl.BlockSpec((_BLK, D), lambda i: (i, 0)),
    out_shape=jax.ShapeDtypeStruct((N, D), jnp.float32),
)


def kernel(x, edge_index, edge_weight, W1l, W1r, b1, W2l, W2r, b2):
    src = edge_index[0]
    dst = edge_index[1]
    y1, r1 = _tc_pre(x, W1l, W1r, b1.reshape(1, D))
    p1x, p1c = _sc_agg(y1, src, dst, edge_weight)
    y2, r2 = _tc_mid(p1x, p1c, r1, W2l, W2r, b2.reshape(1, D))
    p2x, p2c = _sc_agg(y2, src, dst, edge_weight)
    return _tc_post(p2x, p2c, r2)
